# baseline (speedup 1.0000x reference)
import functools

import jax
import jax.numpy as jnp
from jax import lax
from jax.experimental import pallas as pl
from jax.experimental.pallas import tpu as pltpu
from jax.experimental.pallas import tpu_sc as plsc

HIDDEN = 128
B, T = 4, 2048
EPS = 1e-6
NTOK = B * T
NW = 32
ROWS_PER_W = NTOK // NW
WPB = T // ROWS_PER_W
L = 16
NCH = HIDDEN // L
GCH = 128
NG = ROWS_PER_W // GCH


def _rsqrt16(x):
    i = lax.bitcast_convert_type(x, jnp.int32)
    i = jnp.int32(0x5F3759DF) - lax.shift_right_arithmetic(i, 1)
    y = lax.bitcast_convert_type(i, jnp.float32)
    xh = x * 0.5
    for _ in range(3):
        y = y * (1.5 - xh * y * y)
    return y


@functools.partial(
    pl.kernel,
    out_type=jax.ShapeDtypeStruct((B, T, HIDDEN), jnp.float32),
    mesh=plsc.VectorSubcoreMesh(core_axis_name="c", subcore_axis_name="s"),
    scratch_types=[
        pltpu.VMEM((ROWS_PER_W,), jnp.int32),
        pltpu.VMEM((ROWS_PER_W,), jnp.int32),
        pltpu.VMEM((2, HIDDEN), jnp.float32),
        pltpu.VMEM((HIDDEN,), jnp.float32),
        pltpu.VMEM((ROWS_PER_W, HIDDEN), jnp.float32),
        pltpu.SemaphoreType.DMA,
        pltpu.SemaphoreType.DMA,
        pltpu.SemaphoreType.DMA,
        pltpu.SemaphoreType.DMA,
    ],
)
def _emb_kernel(word_hbm, ids_hbm, tt_hbm, type_hbm, w_hbm, out_hbm,
                idx_v, tt_v, type_v, w_v, rows_v,
                sem0, sem1, sem_aux, sem_st):
    wid = lax.axis_index("s") * 2 + lax.axis_index("c")
    brow = wid // WPB
    tok0 = (wid % WPB) * ROWS_PER_W
    sems = (sem0, sem1)

    pltpu.sync_copy(ids_hbm.at[brow, pl.ds(tok0, ROWS_PER_W)], idx_v)
    gathers = [
        pltpu.async_copy(
            word_hbm.at[idx_v.at[pl.ds(g * GCH, GCH)]],
            rows_v.at[pl.ds(g * GCH, GCH)],
            sems[g],
        )
        for g in range(NG)
    ]

    c_tt = pltpu.async_copy(tt_hbm.at[brow, pl.ds(tok0, ROWS_PER_W)], tt_v, sem_aux)
    c_ty = pltpu.async_copy(type_hbm, type_v, sem_aux)
    c_w = pltpu.async_copy(w_hbm, w_v, sem_aux)
    c_tt.wait()
    c_ty.wait()
    c_w.wait()

    t0 = [type_v[0, pl.ds(c * L, L)] for c in range(NCH)]
    td = [type_v[1, pl.ds(c * L, L)] - type_v[0, pl.ds(c * L, L)]
          for c in range(NCH)]
    wsc = [w_v[pl.ds(c * L, L)] * float(HIDDEN) ** 0.5 for c in range(NCH)]
    HEPS = float(HIDDEN) * EPS

    lanes = lax.iota(jnp.int32, L)
    perms3 = [lax.bitwise_xor(lanes, jnp.int32(k)) for k in (1, 2, 4)]
    perm8 = lax.bitwise_xor(lanes, jnp.int32(8))
    h0 = (-lax.shift_right_arithmetic(lanes - 8, 31)).astype(jnp.float32)
    h1 = 1.0 - h0

    for gth in gathers:
        gth.wait()

    @plsc.parallel_loop(0, ROWS_PER_W // L)
    def group_body(g):
        rbase = g * L
        ttf16 = tt_v[pl.ds(rbase, L)].astype(jnp.float32)
        for rp in range(L // 2):
            r0 = rbase + 2 * rp
            xs2 = []
            acc2 = []
            for rr in (0, 1):
                r = r0 + rr
                ttf = jnp.broadcast_to(ttf16[2 * rp + rr], (L,))
                xs = []
                acc = None
                for c in range(NCH):
                    xc = rows_v[r, pl.ds(c * L, L)] + (t0[c] + ttf * td[c])
                    xs.append(xc)
                    sq = xc * xc
                    acc = sq if acc is None else acc + sq
                xs2.append(xs)
                acc2.append(acc)
            sa, sb = acc2
            for p in perms3:
                sa = sa + sa.at[p].get(mode="promise_in_bounds")
                sb = sb + sb.at[p].get(mode="promise_in_bounds")
            sa = sa + sa.at[perm8].get(mode="promise_in_bounds")
            sb = sb + sb.at[perm8].get(mode="promise_in_bounds")
            s2 = sa * h0 + sb * h1
            scale2 = _rsqrt16(s2 + HEPS)
            sc_a = jnp.broadcast_to(scale2[0], (L,))
            sc_b = jnp.broadcast_to(scale2[8], (L,))
            for rr, scv in ((0, sc_a), (1, sc_b)):
                r = r0 + rr
                xs = xs2[rr]
                for c in range(NCH):
                    rows_v[r, pl.ds(c * L, L)] = (xs[c] * scv) * wsc[c]

        pltpu.async_copy(
            rows_v.at[pl.ds(rbase, L)],
            out_hbm.at[brow, pl.ds(tok0 + rbase, L)],
            sem_st,
        )

    for gg in range(ROWS_PER_W // L):
        pltpu.make_async_copy(
            rows_v.at[pl.ds(gg * L, L)],
            out_hbm.at[brow, pl.ds(tok0 + gg * L, L)],
            sem_st,
        ).wait()


def kernel(input_ids, token_type_ids, word_emb, type_emb, ln_weight):
    ids = input_ids.astype(jnp.int32)
    tt = token_type_ids.astype(jnp.int32)
    return _emb_kernel(word_emb, ids, tt, type_emb, ln_weight)


# speedup vs baseline: 1.0697x; 1.0697x over previous
import functools

import jax
import jax.numpy as jnp
from jax import lax
from jax.experimental import pallas as pl
from jax.experimental.pallas import tpu as pltpu
from jax.experimental.pallas import tpu_sc as plsc

HIDDEN = 128
B, T = 4, 2048
EPS = 1e-6
NTOK = B * T
NW = 32
ROWS_PER_W = NTOK // NW
WPB = T // ROWS_PER_W
L = 16
NCH = HIDDEN // L
GCH = 128
NG = ROWS_PER_W // GCH


def _rsqrt16(x):
    i = lax.bitcast_convert_type(x, jnp.int32)
    i = jnp.int32(0x5F3759DF) - lax.shift_right_arithmetic(i, 1)
    y = lax.bitcast_convert_type(i, jnp.float32)
    xh = x * 0.5
    for _ in range(3):
        y = y * (1.5 - xh * y * y)
    return y


@functools.partial(
    pl.kernel,
    out_type=jax.ShapeDtypeStruct((B, T, HIDDEN), jnp.float32),
    mesh=plsc.VectorSubcoreMesh(core_axis_name="c", subcore_axis_name="s"),
    scratch_types=[
        pltpu.VMEM((ROWS_PER_W,), jnp.int32),
        pltpu.VMEM((ROWS_PER_W,), jnp.int32),
        pltpu.VMEM((2, HIDDEN), jnp.float32),
        pltpu.VMEM((HIDDEN,), jnp.float32),
        pltpu.VMEM((ROWS_PER_W, HIDDEN), jnp.float32),
        pltpu.SemaphoreType.DMA,
        pltpu.SemaphoreType.DMA,
        pltpu.SemaphoreType.DMA,
        pltpu.SemaphoreType.DMA,
    ],
)
def _emb_kernel(word_hbm, ids_hbm, tt_hbm, type_hbm, w_hbm, out_hbm,
                idx_v, tt_v, type_v, w_v, rows_v,
                sem0, sem1, sem_aux, sem_st):
    wid = lax.axis_index("s") * 2 + lax.axis_index("c")
    brow = wid // WPB
    tok0 = (wid % WPB) * ROWS_PER_W
    sems = (sem0, sem1)

    pltpu.sync_copy(ids_hbm.at[brow, pl.ds(tok0, ROWS_PER_W)], idx_v)
    gathers = [
        pltpu.async_copy(
            word_hbm.at[idx_v.at[pl.ds(g * GCH, GCH)]],
            rows_v.at[pl.ds(g * GCH, GCH)],
            sems[g],
        )
        for g in range(NG)
    ]

    c_tt = pltpu.async_copy(tt_hbm.at[brow, pl.ds(tok0, ROWS_PER_W)], tt_v, sem_aux)
    c_ty = pltpu.async_copy(type_hbm, type_v, sem_aux)
    c_w = pltpu.async_copy(w_hbm, w_v, sem_aux)
    c_tt.wait()
    c_ty.wait()
    c_w.wait()

    t0 = [type_v[0, pl.ds(c * L, L)] for c in range(NCH)]
    td = [type_v[1, pl.ds(c * L, L)] - type_v[0, pl.ds(c * L, L)]
          for c in range(NCH)]
    wsc = [w_v[pl.ds(c * L, L)] * float(HIDDEN) ** 0.5 for c in range(NCH)]
    HEPS = float(HIDDEN) * EPS

    lanes = lax.iota(jnp.int32, L)
    perms3 = [lax.bitwise_xor(lanes, jnp.int32(k)) for k in (1, 2, 4)]
    perm8 = lax.bitwise_xor(lanes, jnp.int32(8))
    h0 = (-lax.shift_right_arithmetic(lanes - 8, 31)).astype(jnp.float32)
    h1 = 1.0 - h0

    QR = 64
    st_descs = [
        pltpu.make_async_copy(
            rows_v.at[pl.ds(q * QR, QR)],
            out_hbm.at[brow, pl.ds(tok0 + q * QR, QR)],
            sem_st,
        )
        for q in range(3)
    ]

    def group_body(g, carry):
        @pl.when(g == 8)
        def _wait_second_gather():
            gathers[1].wait()

        rbase = g * L
        ttf16 = tt_v[pl.ds(rbase, L)].astype(jnp.float32)
        for rp in range(L // 2):
            r0 = rbase + 2 * rp
            xs2 = []
            acc2 = []
            for rr in (0, 1):
                r = r0 + rr
                ttf = jnp.broadcast_to(ttf16[2 * rp + rr], (L,))
                xs = []
                acc = None
                for c in range(NCH):
                    xc = rows_v[r, pl.ds(c * L, L)] + (t0[c] + ttf * td[c])
                    xs.append(xc)
                    sq = xc * xc
                    acc = sq if acc is None else acc + sq
                xs2.append(xs)
                acc2.append(acc)
            sa, sb = acc2
            for p in perms3:
                sa = sa + sa.at[p].get(mode="promise_in_bounds")
                sb = sb + sb.at[p].get(mode="promise_in_bounds")
            sa = sa + sa.at[perm8].get(mode="promise_in_bounds")
            sb = sb + sb.at[perm8].get(mode="promise_in_bounds")
            s2 = sa * h0 + sb * h1
            scale2 = _rsqrt16(s2 + HEPS)
            sc_a = jnp.broadcast_to(scale2[0], (L,))
            sc_b = jnp.broadcast_to(scale2[8], (L,))
            for rr, scv in ((0, sc_a), (1, sc_b)):
                r = r0 + rr
                xs = xs2[rr]
                for c in range(NCH):
                    rows_v[r, pl.ds(c * L, L)] = (xs[c] * scv) * wsc[c]

        for q in range(3):
            @pl.when(g == 4 * q + 3)
            def _store_quarter(q=q):
                st_descs[q].start()

        return carry

    gathers[0].wait()
    lax.fori_loop(0, ROWS_PER_W // L, group_body, 0)
    pltpu.sync_copy(rows_v.at[pl.ds(3 * QR, QR)],
                    out_hbm.at[brow, pl.ds(tok0 + 3 * QR, QR)])
    for q in range(3):
        st_descs[q].wait()


def kernel(input_ids, token_type_ids, word_emb, type_emb, ln_weight):
    ids = input_ids.astype(jnp.int32)
    tt = token_type_ids.astype(jnp.int32)
    return _emb_kernel(word_emb, ids, tt, type_emb, ln_weight)
